# baseline (speedup 1.0000x reference)
import functools

import jax
import jax.numpy as jnp
import ml_dtypes
import numpy as np
from jax import lax
from jax.experimental import pallas as pl
from jax.experimental.pallas import tpu as pltpu
from jax.experimental.pallas import tpu_sc as plsc

_POLAR = (64, 1024)
_CART = (512, 512)
_CDGS = 3.0
_B, _C = 4, 32

_PLANES = _B * _C
_QN = 256 * 256
_WW = 272
_TR = 81
_BOXW = 16
_WC0 = (752, 504, 0, 256)

_K = 2048
_ROWS = _K // 256
_NCHUNK = _QN // _K
_PPASS = 4
_NPASS = 4


def _build_static():
    yy_org, xx_org = np.meshgrid(np.arange(_CART[0]), np.arange(_CART[1]),
                                 indexing='ij')
    yy = (yy_org - _CART[0] / 2.0 + 0.5).astype(np.float32)
    xx = (xx_org - _CART[1] / 2.0 + 0.5).astype(np.float32)
    depth = np.sqrt(xx ** 2 + yy ** 2)
    phi = np.pi - np.arctan2(yy, xx)
    index_y = depth / (_CART[0] / 2.0 * np.sqrt(2.0)) * (_POLAR[0] + _CDGS) - _CDGS
    index_x = phi / np.pi / 2.0 * _POLAR[1]
    mask = index_y > 0
    gx = (index_x / _POLAR[1] * 2.0 - 1.0).astype(np.float32)
    gy = (-(index_y / _POLAR[0] * 2.0 - 1.0)).astype(np.float32)
    ix = ((gx + np.float32(1.0)) * np.float32(0.5) * np.float32(_POLAR[1] - 1))
    iy = ((gy + np.float32(1.0)) * np.float32(0.5) * np.float32(_POLAR[0] - 1))
    ix0 = np.floor(ix)
    iy0 = np.floor(iy)
    wx0 = (np.float32(1.0) - (ix - ix0)).astype(np.float32)
    wy0 = (np.float32(1.0) - (iy - iy0)).astype(np.float32)

    stream = np.empty((4, _NCHUNK, 2, _K), np.int32)
    for q in range(4):
        r0, c0 = (q // 2) * 256, (q % 2) * 256
        sl = (slice(r0, r0 + 256), slice(c0, c0 + 256))
        wix0 = ix0[sl] - _WC0[q]
        m = mask[sl]
        assert wix0[m].min() >= 0 and wix0[m].max() + 1 < _WW
        idx = (iy0[sl].astype(np.int64) * 512 + wix0.astype(np.int64)).astype(np.int32)
        by0, bx0 = 240 if q < 2 else 256, 240 if q % 2 == 0 else 256
        box_idx = ((64 + yy_org[sl] - by0) * 512 + (xx_org[sl] - bx0)).astype(np.int32)
        idx = np.where(m, idx, box_idx).reshape(-1)
        wxq = np.where(m, wx0[sl], np.float32(1.0)).reshape(-1).astype(np.float32)
        wyq = np.where(m, wy0[sl], np.float32(1.0)).reshape(-1).astype(np.float32)
        wxb = wxq.astype(ml_dtypes.bfloat16).view(np.uint16).astype(np.uint32)
        wyb = wyq.astype(ml_dtypes.bfloat16).view(np.uint16).astype(np.uint32)
        w = ((wxb << 16) | wyb).view(np.int32)
        stream[q, :, 0, :] = idx.reshape(-1, _K)
        stream[q, :, 1, :] = w.reshape(-1, _K)
    return stream.reshape(4 * _NCHUNK, 2 * _K)


_STREAM_NP = _build_static()


@functools.cache
def _make_sc_resample():
    mesh = plsc.VectorSubcoreMesh(core_axis_name="c", subcore_axis_name="s")
    return functools.partial(
        pl.kernel,
        mesh=mesh,
        out_type=jax.ShapeDtypeStruct((_PLANES, _CART[0], _CART[1]), jnp.float32),
        scratch_types=[
            [pltpu.VMEM((_TR, _WW), jnp.float32) for _ in range(_PPASS)],
            [pltpu.VMEM((2 * _K,), jnp.int32) for _ in range(2)],
            [[pltpu.VMEM((_ROWS, 256), jnp.float32) for _ in range(2)]
             for _ in range(_PPASS)],
            [pltpu.SemaphoreType.DMA for _ in range(2)],
            [pltpu.SemaphoreType.DMA for _ in range(2)],
        ],
        compiler_params=pltpu.CompilerParams(needs_layout_passes=False,
                                             use_tc_tiling_on_sc=False),
    )(_sc_resample_body)


def _sc_resample_body(polar_hbm, ref_hbm, stream_hbm, out_hbm,
                      tables_v, inbufs_v, obufs_v, sem_in, sem_out):
    wid = lax.axis_index("s") * 2 + lax.axis_index("c")
    q = wid % 4
    grp = wid // 4
    qr0 = (q // 2) * 256
    qc0 = (q % 2) * 256
    wc0 = jnp.where(q == 0, _WC0[0],
                    jnp.where(q == 1, _WC0[1],
                              jnp.where(q == 2, _WC0[2], _WC0[3])))
    by0 = jnp.where(q < 2, 240, 256)
    bx0 = jnp.where(q % 2 == 0, 240, 256)

    zeros = jnp.zeros((16,), jnp.float32)
    for t in range(_PPASS):
        def zrow(r, carry, t=t):
            def zcol(ci, carry2):
                tables_v[t][r, pl.ds(ci * 16, 16)] = zeros
                return carry2
            return lax.fori_loop(0, _WW // 16, zcol, carry)
        lax.fori_loop(64, _TR, zrow, 0)

    def start_in(cix, b):
        pltpu.async_copy(stream_hbm.at[q * _NCHUNK + cix], inbufs_v[b],
                         sem_in[b])

    def wait_in(b):
        pltpu.make_async_copy(stream_hbm.at[0], inbufs_v[b], sem_in[b]).wait()

    def drain_out(b):
        for t in range(_PPASS):
            pltpu.make_async_copy(
                out_hbm.at[0, pl.ds(0, _ROWS), pl.ds(0, 256)],
                obufs_v[t][b], sem_out[b]).wait()

    def pass_body(ps, carry):
        pbase = grp * (_PPASS * _NPASS) + ps * _PPASS
        for t in range(_PPASS):
            pltpu.sync_copy(polar_hbm.at[pbase + t, :, pl.ds(wc0, _WW)],
                            tables_v[t].at[pl.ds(0, _POLAR[0])])
            pltpu.sync_copy(ref_hbm.at[pbase + t, pl.ds(by0, _BOXW),
                                       pl.ds(bx0, _BOXW)],
                            tables_v[t].at[pl.ds(64, _BOXW), pl.ds(0, _BOXW)])

        start_in(0, 0)
        start_in(1, 1)

        def chunk_pair_body(half, carry2):
            for b in range(2):
                cix = half * 2 + b
                wait_in(b)
                pl.when(cix >= 2)(lambda b=b: drain_out(b))
                inbuf = inbufs_v[b]
                y0 = qr0 + cix * _ROWS
                for t in range(_PPASS):
                    table_v = tables_v[t]
                    ob_v = obufs_v[t][b]

                    @plsc.parallel_loop(0, _K, 16, unroll=8)
                    def px_body(o):
                        idx = inbuf[pl.ds(o, 16)]
                        w = inbuf[pl.ds(_K + o, 16)]
                        row = lax.shift_right_logical(idx, 9)
                        col = jnp.bitwise_and(idx, 511)
                        row1 = row + 1
                        col1 = col + 1
                        v00 = plsc.load_gather(table_v, [row, col])
                        v01 = plsc.load_gather(table_v, [row, col1])
                        v10 = plsc.load_gather(table_v, [row1, col])
                        v11 = plsc.load_gather(table_v, [row1, col1])
                        wx0 = plsc.bitcast(
                            jnp.bitwise_and(w, jnp.int32(-65536)), jnp.float32)
                        wy0 = plsc.bitcast(lax.shift_left(w, 16), jnp.float32)
                        wx1 = 1.0 - wx0
                        wy1 = 1.0 - wy0
                        r = (wy0 * (wx0 * v00 + wx1 * v01)
                             + wy1 * (wx0 * v10 + wx1 * v11))
                        ob_v[o // 256, pl.ds(o % 256, 16)] = r

                    pltpu.async_copy(
                        ob_v,
                        out_hbm.at[pbase + t, pl.ds(y0, _ROWS), pl.ds(qc0, 256)],
                        sem_out[b])
                pl.when(cix + 2 < _NCHUNK)(lambda cix=cix, b=b: start_in(cix + 2, b))
            return carry2

        lax.fori_loop(0, _NCHUNK // 2, chunk_pair_body, 0)
        drain_out(0)
        drain_out(1)
        return carry

    lax.fori_loop(0, _NPASS, pass_body, 0)


def kernel(polar_feat, ref_feat):
    polar3 = polar_feat.reshape(_PLANES, _POLAR[0], _POLAR[1])
    ref3 = ref_feat.reshape(_PLANES, _CART[0], _CART[1])
    stream = jnp.asarray(_STREAM_NP)
    out = _make_sc_resample()(polar3, ref3, stream)
    return out.reshape(_B, _C, _CART[0], _CART[1])


# speedup vs baseline: 1.9305x; 1.9305x over previous
import functools

import jax
import jax.numpy as jnp
import ml_dtypes
import numpy as np
from jax import lax
from jax.experimental import pallas as pl
from jax.experimental.pallas import tpu as pltpu
from jax.experimental.pallas import tpu_sc as plsc

_POLAR = (64, 1024)
_CART = (512, 512)
_CDGS = 3.0
_B, _C = 4, 32

_PLANES = _B * _C
_QN = 256 * 256
_WW = 264
_WEDGE = _POLAR[0] * _WW
_BOXW = 16
_BOX = _BOXW * _BOXW
_PAD = 272
_TW = _WEDGE + _BOX + _PAD
_WC0 = (760, 508, 0, 256)

_K = 2048
_ROWS = _K // 256
_NCHUNK = _QN // _K
_PPASS = 4
_NPASS = 4


def _build_static():
    yy_org, xx_org = np.meshgrid(np.arange(_CART[0]), np.arange(_CART[1]),
                                 indexing='ij')
    yy = (yy_org - _CART[0] / 2.0 + 0.5).astype(np.float32)
    xx = (xx_org - _CART[1] / 2.0 + 0.5).astype(np.float32)
    depth = np.sqrt(xx ** 2 + yy ** 2)
    phi = np.pi - np.arctan2(yy, xx)
    index_y = depth / (_CART[0] / 2.0 * np.sqrt(2.0)) * (_POLAR[0] + _CDGS) - _CDGS
    index_x = phi / np.pi / 2.0 * _POLAR[1]
    mask = index_y > 0
    gx = (index_x / _POLAR[1] * 2.0 - 1.0).astype(np.float32)
    gy = (-(index_y / _POLAR[0] * 2.0 - 1.0)).astype(np.float32)
    ix = ((gx + np.float32(1.0)) * np.float32(0.5) * np.float32(_POLAR[1] - 1))
    iy = ((gy + np.float32(1.0)) * np.float32(0.5) * np.float32(_POLAR[0] - 1))
    ix0 = np.floor(ix)
    iy0 = np.floor(iy)
    wx0 = (np.float32(1.0) - (ix - ix0)).astype(np.float32)
    wy0 = (np.float32(1.0) - (iy - iy0)).astype(np.float32)

    stream = np.empty((4, _NCHUNK, 2, _K), np.int32)
    for q in range(4):
        r0, c0 = (q // 2) * 256, (q % 2) * 256
        sl = (slice(r0, r0 + 256), slice(c0, c0 + 256))
        wix0 = ix0[sl] - _WC0[q]
        m = mask[sl]
        assert wix0[m].min() >= 0 and wix0[m].max() + 1 < _WW
        idx = (iy0[sl].astype(np.int64) * _WW + wix0.astype(np.int64)).astype(np.int32)
        by0, bx0 = 240 if q < 2 else 256, 240 if q % 2 == 0 else 256
        box_idx = (_WEDGE + (yy_org[sl] - by0) * _BOXW
                   + (xx_org[sl] - bx0)).astype(np.int32)
        idx = np.where(m, idx, box_idx).reshape(-1)
        wxq = np.where(m, wx0[sl], np.float32(1.0)).reshape(-1).astype(np.float32)
        wyq = np.where(m, wy0[sl], np.float32(1.0)).reshape(-1).astype(np.float32)
        wxb = wxq.astype(ml_dtypes.bfloat16).view(np.uint16).astype(np.uint32)
        wyb = wyq.astype(ml_dtypes.bfloat16).view(np.uint16).astype(np.uint32)
        w = ((wxb << 16) | wyb).view(np.int32)
        stream[q, :, 0, :] = idx.reshape(-1, _K)
        stream[q, :, 1, :] = w.reshape(-1, _K)
    return stream.reshape(4 * _NCHUNK, 2 * _K)


_STREAM_NP = _build_static()


@functools.cache
def _make_sc_resample():
    mesh = plsc.VectorSubcoreMesh(core_axis_name="c", subcore_axis_name="s")
    return functools.partial(
        pl.kernel,
        mesh=mesh,
        out_type=jax.ShapeDtypeStruct((_PLANES, _CART[0], _CART[1]), jnp.float32),
        scratch_types=[
            [pltpu.VMEM((_TW,), jnp.float32) for _ in range(_PPASS)],
            [pltpu.VMEM((2 * _K,), jnp.int32) for _ in range(2)],
            [[pltpu.VMEM((_ROWS, 256), jnp.float32) for _ in range(2)]
             for _ in range(_PPASS)],
            [pltpu.SemaphoreType.DMA for _ in range(2)],
            [pltpu.SemaphoreType.DMA for _ in range(2)],
        ],
        compiler_params=pltpu.CompilerParams(needs_layout_passes=False),
    )(_sc_resample_body)


def _sc_resample_body(wedges_hbm, boxes_hbm, stream_hbm, out_hbm,
                      tables_v, inbufs_v, obufs_v, sem_in, sem_out):
    wid = lax.axis_index("s") * 2 + lax.axis_index("c")
    q = wid % 4
    grp = wid // 4
    qr0 = (q // 2) * 256
    qc0 = (q % 2) * 256

    zeros = jnp.zeros((16,), jnp.float32)
    for t in range(_PPASS):
        def zbody(i, carry, t=t):
            tables_v[t][pl.ds(_WEDGE + _BOX + i * 16, 16)] = zeros
            return carry
        lax.fori_loop(0, _PAD // 16, zbody, 0)

    def start_in(cix, b):
        pltpu.async_copy(stream_hbm.at[q * _NCHUNK + cix], inbufs_v[b],
                         sem_in[b])

    def wait_in(b):
        pltpu.make_async_copy(stream_hbm.at[0], inbufs_v[b], sem_in[b]).wait()

    def drain_out(b):
        for t in range(_PPASS):
            pltpu.make_async_copy(
                out_hbm.at[0, pl.ds(0, _ROWS), pl.ds(0, 256)],
                obufs_v[t][b], sem_out[b]).wait()

    def pass_body(ps, carry):
        pbase = grp * (_PPASS * _NPASS) + ps * _PPASS
        for t in range(_PPASS):
            pltpu.sync_copy(wedges_hbm.at[q, pbase + t],
                            tables_v[t].at[pl.ds(0, _WEDGE)])
            pltpu.sync_copy(boxes_hbm.at[q, pbase + t],
                            tables_v[t].at[pl.ds(_WEDGE, _BOX)])

        start_in(0, 0)
        start_in(1, 1)

        def chunk_pair_body(half, carry2):
            for b in range(2):
                cix = half * 2 + b
                wait_in(b)
                pl.when(cix >= 2)(lambda b=b: drain_out(b))
                inbuf = inbufs_v[b]
                y0 = qr0 + cix * _ROWS
                for t in range(_PPASS):
                    table_v = tables_v[t]
                    ob_v = obufs_v[t][b]

                    @plsc.parallel_loop(0, _K, 16, unroll=8)
                    def px_body(o):
                        idx = inbuf[pl.ds(o, 16)]
                        w = inbuf[pl.ds(_K + o, 16)]
                        v00 = plsc.load_gather(table_v, [idx])
                        v01 = plsc.load_gather(table_v, [idx + 1])
                        v10 = plsc.load_gather(table_v, [idx + _WW])
                        v11 = plsc.load_gather(table_v, [idx + (_WW + 1)])
                        wx0 = plsc.bitcast(
                            jnp.bitwise_and(w, jnp.int32(-65536)), jnp.float32)
                        wy0 = plsc.bitcast(lax.shift_left(w, 16), jnp.float32)
                        wx1 = 1.0 - wx0
                        wy1 = 1.0 - wy0
                        r = (wy0 * (wx0 * v00 + wx1 * v01)
                             + wy1 * (wx0 * v10 + wx1 * v11))
                        ob_v[o // 256, pl.ds(o % 256, 16)] = r

                    pltpu.async_copy(
                        ob_v,
                        out_hbm.at[pbase + t, pl.ds(y0, _ROWS), pl.ds(qc0, 256)],
                        sem_out[b])
                pl.when(cix + 2 < _NCHUNK)(lambda cix=cix, b=b: start_in(cix + 2, b))
            return carry2

        lax.fori_loop(0, _NCHUNK // 2, chunk_pair_body, 0)
        drain_out(0)
        drain_out(1)
        return carry

    lax.fori_loop(0, _NPASS, pass_body, 0)


def kernel(polar_feat, ref_feat):
    polar3 = polar_feat.reshape(_PLANES, _POLAR[0], _POLAR[1])
    ref3 = ref_feat.reshape(_PLANES, _CART[0], _CART[1])
    wedges = jnp.stack(
        [polar3[:, :, c0:c0 + _WW].reshape(_PLANES, _WEDGE) for c0 in _WC0])
    boxes = jnp.stack(
        [ref3[:, by0:by0 + _BOXW, bx0:bx0 + _BOXW].reshape(_PLANES, _BOX)
         for by0, bx0 in ((240, 240), (240, 256), (256, 240), (256, 256))])
    stream = jnp.asarray(_STREAM_NP)
    out = _make_sc_resample()(wedges, boxes, stream)
    return out.reshape(_B, _C, _CART[0], _CART[1])


# speedup vs baseline: 1.9719x; 1.0214x over previous
import functools

import jax
import jax.numpy as jnp
import ml_dtypes
import numpy as np
from jax import lax
from jax.experimental import pallas as pl
from jax.experimental.pallas import tpu as pltpu
from jax.experimental.pallas import tpu_sc as plsc

_POLAR = (64, 1024)
_CART = (512, 512)
_CDGS = 3.0
_B, _C = 4, 32

_PLANES = _B * _C
_QN = 256 * 256
_WW = 264
_WEDGE = _POLAR[0] * _WW
_BOXW = 16
_BOX = _BOXW * _BOXW
_PAD = 272
_TW = _WEDGE + _BOX + _PAD
_WC0 = (760, 508, 0, 256)

_K = 2048
_ROWS = _K // 256
_NCHUNK = _QN // _K
_PPASS = 4
_NPASS = 4


def _build_static():
    yy_org, xx_org = np.meshgrid(np.arange(_CART[0]), np.arange(_CART[1]),
                                 indexing='ij')
    yy = (yy_org - _CART[0] / 2.0 + 0.5).astype(np.float32)
    xx = (xx_org - _CART[1] / 2.0 + 0.5).astype(np.float32)
    depth = np.sqrt(xx ** 2 + yy ** 2)
    phi = np.pi - np.arctan2(yy, xx)
    index_y = depth / (_CART[0] / 2.0 * np.sqrt(2.0)) * (_POLAR[0] + _CDGS) - _CDGS
    index_x = phi / np.pi / 2.0 * _POLAR[1]
    mask = index_y > 0
    gx = (index_x / _POLAR[1] * 2.0 - 1.0).astype(np.float32)
    gy = (-(index_y / _POLAR[0] * 2.0 - 1.0)).astype(np.float32)
    ix = ((gx + np.float32(1.0)) * np.float32(0.5) * np.float32(_POLAR[1] - 1))
    iy = ((gy + np.float32(1.0)) * np.float32(0.5) * np.float32(_POLAR[0] - 1))
    ix0 = np.floor(ix)
    iy0 = np.floor(iy)
    wx0 = (np.float32(1.0) - (ix - ix0)).astype(np.float32)
    wy0 = (np.float32(1.0) - (iy - iy0)).astype(np.float32)

    stream = np.empty((4, _NCHUNK, 2, _K), np.int32)
    for q in range(4):
        r0, c0 = (q // 2) * 256, (q % 2) * 256
        sl = (slice(r0, r0 + 256), slice(c0, c0 + 256))
        wix0 = ix0[sl] - _WC0[q]
        m = mask[sl]
        assert wix0[m].min() >= 0 and wix0[m].max() + 1 < _WW
        idx = (iy0[sl].astype(np.int64) * _WW + wix0.astype(np.int64)).astype(np.int32)
        by0, bx0 = 240 if q < 2 else 256, 240 if q % 2 == 0 else 256
        box_idx = (_WEDGE + (yy_org[sl] - by0) * _BOXW
                   + (xx_org[sl] - bx0)).astype(np.int32)
        idx = np.where(m, idx, box_idx).reshape(-1)
        wxq = np.where(m, wx0[sl], np.float32(1.0)).reshape(-1).astype(np.float32)
        wyq = np.where(m, wy0[sl], np.float32(1.0)).reshape(-1).astype(np.float32)
        wxb = wxq.astype(ml_dtypes.bfloat16).view(np.uint16).astype(np.uint32)
        wyb = wyq.astype(ml_dtypes.bfloat16).view(np.uint16).astype(np.uint32)
        w = ((wxb << 16) | wyb).view(np.int32)
        stream[q, :, 0, :] = idx.reshape(-1, _K)
        stream[q, :, 1, :] = w.reshape(-1, _K)
    return stream.reshape(4 * _NCHUNK, 2 * _K)


_STREAM_NP = _build_static()


@functools.cache
def _make_sc_resample():
    mesh = plsc.VectorSubcoreMesh(core_axis_name="c", subcore_axis_name="s")
    return functools.partial(
        pl.kernel,
        mesh=mesh,
        out_type=jax.ShapeDtypeStruct((_PLANES, _CART[0], _CART[1]), jnp.float32),
        scratch_types=[
            [pltpu.VMEM((_TW,), jnp.float32) for _ in range(_PPASS)],
            [pltpu.VMEM((2 * _K,), jnp.int32) for _ in range(2)],
            [[pltpu.VMEM((_ROWS, 256), jnp.float32) for _ in range(2)]
             for _ in range(_PPASS)],
            [pltpu.SemaphoreType.DMA for _ in range(2)],
            [pltpu.SemaphoreType.DMA for _ in range(2)],
        ],
        compiler_params=pltpu.CompilerParams(needs_layout_passes=False),
    )(_sc_resample_body)


def _sc_resample_body(tabs_hbm, stream_hbm, out_hbm,
                      tables_v, inbufs_v, obufs_v, sem_in, sem_out):
    wid = lax.axis_index("s") * 2 + lax.axis_index("c")
    q = wid % 4
    grp = wid // 4
    qr0 = (q // 2) * 256
    qc0 = (q % 2) * 256

    def start_in(cix, b):
        pltpu.async_copy(stream_hbm.at[q * _NCHUNK + cix], inbufs_v[b],
                         sem_in[b])

    def wait_in(b):
        pltpu.make_async_copy(stream_hbm.at[0], inbufs_v[b], sem_in[b]).wait()

    def drain_out(b):
        for t in range(_PPASS):
            pltpu.make_async_copy(
                out_hbm.at[0, pl.ds(0, _ROWS), pl.ds(0, 256)],
                obufs_v[t][b], sem_out[b]).wait()

    def pass_body(ps, carry):
        pbase = grp * (_PPASS * _NPASS) + ps * _PPASS
        for t in range(_PPASS):
            pltpu.sync_copy(tabs_hbm.at[q, pbase + t], tables_v[t])

        start_in(0, 0)
        start_in(1, 1)

        def chunk_pair_body(half, carry2):
            for b in range(2):
                cix = half * 2 + b
                wait_in(b)
                pl.when(cix >= 2)(lambda b=b: drain_out(b))
                inbuf = inbufs_v[b]
                y0 = qr0 + cix * _ROWS
                for t in range(_PPASS):
                    table_v = tables_v[t]
                    ob_v = obufs_v[t][b]

                    @plsc.parallel_loop(0, _K, 16, unroll=8)
                    def px_body(o):
                        idx = inbuf[pl.ds(o, 16)]
                        w = inbuf[pl.ds(_K + o, 16)]
                        v00 = plsc.load_gather(table_v, [idx])
                        v01 = plsc.load_gather(table_v, [idx + 1])
                        v10 = plsc.load_gather(table_v, [idx + _WW])
                        v11 = plsc.load_gather(table_v, [idx + (_WW + 1)])
                        wx0 = plsc.bitcast(
                            jnp.bitwise_and(w, jnp.int32(-65536)), jnp.float32)
                        wy0 = plsc.bitcast(lax.shift_left(w, 16), jnp.float32)
                        wx1 = 1.0 - wx0
                        wy1 = 1.0 - wy0
                        r = (wy0 * (wx0 * v00 + wx1 * v01)
                             + wy1 * (wx0 * v10 + wx1 * v11))
                        ob_v[o // 256, pl.ds(o % 256, 16)] = r

                    pltpu.async_copy(
                        ob_v,
                        out_hbm.at[pbase + t, pl.ds(y0, _ROWS), pl.ds(qc0, 256)],
                        sem_out[b])
                pl.when(cix + 2 < _NCHUNK)(lambda cix=cix, b=b: start_in(cix + 2, b))
            return carry2

        lax.fori_loop(0, _NCHUNK // 2, chunk_pair_body, 0)
        drain_out(0)
        drain_out(1)
        return carry

    lax.fori_loop(0, _NPASS, pass_body, 0)


def kernel(polar_feat, ref_feat):
    polar3 = polar_feat.reshape(_PLANES, _POLAR[0], _POLAR[1])
    ref3 = ref_feat.reshape(_PLANES, _CART[0], _CART[1])
    wedges = jnp.stack(
        [polar3[:, :, c0:c0 + _WW].reshape(_PLANES, _WEDGE) for c0 in _WC0])
    boxes = jnp.stack(
        [ref3[:, by0:by0 + _BOXW, bx0:bx0 + _BOXW].reshape(_PLANES, _BOX)
         for by0, bx0 in ((240, 240), (240, 256), (256, 240), (256, 256))])
    tabs = jnp.concatenate(
        [wedges, boxes, jnp.zeros((4, _PLANES, _PAD), jnp.float32)], axis=-1)
    stream = jnp.asarray(_STREAM_NP)
    out = _make_sc_resample()(tabs, stream)
    return out.reshape(_B, _C, _CART[0], _CART[1])


# speedup vs baseline: 2.0070x; 1.0178x over previous
import functools

import jax
import jax.numpy as jnp
import ml_dtypes
import numpy as np
from jax import lax
from jax.experimental import pallas as pl
from jax.experimental.pallas import tpu as pltpu
from jax.experimental.pallas import tpu_sc as plsc

_POLAR = (64, 1024)
_CART = (512, 512)
_CDGS = 3.0
_B, _C = 4, 32

_PLANES = _B * _C
_QN = 256 * 256
_WW = 264
_WEDGE = _POLAR[0] * _WW
_BOXW = 16
_BOX = _BOXW * _BOXW
_PAD = 272
_TW = _WEDGE + _BOX + _PAD
_WC0 = (760, 504, 0, 256)

_K = 2048
_ROWS = _K // 256
_NCHUNK = _QN // _K
_PPASS = 4
_NPASS = 4


def _build_static():
    yy_org, xx_org = np.meshgrid(np.arange(_CART[0]), np.arange(_CART[1]),
                                 indexing='ij')
    yy = (yy_org - _CART[0] / 2.0 + 0.5).astype(np.float32)
    xx = (xx_org - _CART[1] / 2.0 + 0.5).astype(np.float32)
    depth = np.sqrt(xx ** 2 + yy ** 2)
    phi = np.pi - np.arctan2(yy, xx)
    index_y = depth / (_CART[0] / 2.0 * np.sqrt(2.0)) * (_POLAR[0] + _CDGS) - _CDGS
    index_x = phi / np.pi / 2.0 * _POLAR[1]
    mask = index_y > 0
    gx = (index_x / _POLAR[1] * 2.0 - 1.0).astype(np.float32)
    gy = (-(index_y / _POLAR[0] * 2.0 - 1.0)).astype(np.float32)
    ix = ((gx + np.float32(1.0)) * np.float32(0.5) * np.float32(_POLAR[1] - 1))
    iy = ((gy + np.float32(1.0)) * np.float32(0.5) * np.float32(_POLAR[0] - 1))
    ix0 = np.floor(ix)
    iy0 = np.floor(iy)
    wx0 = (np.float32(1.0) - (ix - ix0)).astype(np.float32)
    wy0 = (np.float32(1.0) - (iy - iy0)).astype(np.float32)

    stream = np.empty((4, _NCHUNK, 2, _K), np.int32)
    for q in range(4):
        r0, c0 = (q // 2) * 256, (q % 2) * 256
        sl = (slice(r0, r0 + 256), slice(c0, c0 + 256))
        wix0 = ix0[sl] - _WC0[q]
        m = mask[sl]
        assert wix0[m].min() >= 0 and wix0[m].max() + 1 < _WW
        idx = (iy0[sl].astype(np.int64) * _WW + wix0.astype(np.int64)).astype(np.int32)
        by0, bx0 = 240 if q < 2 else 256, 240 if q % 2 == 0 else 256
        box_idx = (_WEDGE + (yy_org[sl] - by0) * _BOXW
                   + (xx_org[sl] - bx0)).astype(np.int32)
        idx = np.where(m, idx, box_idx).reshape(-1)
        wxq = np.where(m, wx0[sl], np.float32(1.0)).reshape(-1).astype(np.float32)
        wyq = np.where(m, wy0[sl], np.float32(1.0)).reshape(-1).astype(np.float32)
        wxb = wxq.astype(ml_dtypes.bfloat16).view(np.uint16).astype(np.uint32)
        wyb = wyq.astype(ml_dtypes.bfloat16).view(np.uint16).astype(np.uint32)
        w = ((wxb << 16) | wyb).view(np.int32)
        stream[q, :, 0, :] = idx.reshape(-1, _K)
        stream[q, :, 1, :] = w.reshape(-1, _K)
    return stream.reshape(4 * _NCHUNK, 2 * _K)


_STREAM_NP = _build_static()


@functools.cache
def _make_sc_resample():
    mesh = plsc.VectorSubcoreMesh(core_axis_name="c", subcore_axis_name="s")
    return functools.partial(
        pl.kernel,
        mesh=mesh,
        out_type=jax.ShapeDtypeStruct((_PLANES, _CART[0], _CART[1]), jnp.float32),
        scratch_types=[
            [pltpu.VMEM((_TW,), jnp.float32) for _ in range(_PPASS)],
            [pltpu.VMEM((2 * _K,), jnp.int32) for _ in range(2)],
            [[pltpu.VMEM((_ROWS, 256), jnp.float32) for _ in range(2)]
             for _ in range(_PPASS)],
            [pltpu.SemaphoreType.DMA for _ in range(2)],
            [pltpu.SemaphoreType.DMA for _ in range(2)],
            pltpu.SemaphoreType.DMA,
        ],
        compiler_params=pltpu.CompilerParams(needs_layout_passes=False),
    )(_sc_resample_body)


def _sc_resample_body(polar_hbm, ref_hbm, stream_hbm, out_hbm,
                      tables_v, inbufs_v, obufs_v, sem_in, sem_out, sem_tab):
    wid = lax.axis_index("s") * 2 + lax.axis_index("c")
    q = wid % 4
    grp = wid // 4
    qr0 = (q // 2) * 256
    qc0 = (q % 2) * 256
    wc0 = jnp.where(q == 0, _WC0[0],
                    jnp.where(q == 1, _WC0[1],
                              jnp.where(q == 2, _WC0[2], _WC0[3])))
    by0 = jnp.where(q < 2, 240, 256)
    bx0 = jnp.where(q % 2 == 0, 240, 256)

    zeros = jnp.zeros((16,), jnp.float32)
    for t in range(_PPASS):
        def zbody(i, carry, t=t):
            tables_v[t][pl.ds(_WEDGE + _BOX + i * 16, 16)] = zeros
            return carry
        lax.fori_loop(0, _PAD // 16, zbody, 0)

    def start_in(cix, b):
        pltpu.async_copy(stream_hbm.at[q * _NCHUNK + cix], inbufs_v[b],
                         sem_in[b])

    def wait_in(b):
        pltpu.make_async_copy(stream_hbm.at[0], inbufs_v[b], sem_in[b]).wait()

    def drain_out(b):
        for t in range(_PPASS):
            pltpu.make_async_copy(
                out_hbm.at[0, pl.ds(0, _ROWS), pl.ds(0, 256)],
                obufs_v[t][b], sem_out[b]).wait()

    def pass_body(ps, carry):
        pbase = grp * (_PPASS * _NPASS) + ps * _PPASS
        for t in range(_PPASS):
            pb = (pbase + t) * (_POLAR[0] * _POLAR[1])
            rb = (pbase + t) * (_CART[0] * _CART[1])

            def trow(r, carry2, t=t, pb=pb):
                pltpu.async_copy(
                    polar_hbm.at[pl.ds(pb + r * _POLAR[1] + wc0, _WW)],
                    tables_v[t].at[pl.ds(r * _WW, _WW)], sem_tab)
                return carry2
            lax.fori_loop(0, _POLAR[0], trow, 0)

            def brow(dy, carry2, t=t, rb=rb):
                pltpu.async_copy(
                    ref_hbm.at[pl.ds(rb + (by0 + dy) * _CART[1] + bx0, _BOXW)],
                    tables_v[t].at[pl.ds(_WEDGE + dy * _BOXW, _BOXW)], sem_tab)
                return carry2
            lax.fori_loop(0, _BOXW, brow, 0)

        start_in(0, 0)
        start_in(1, 1)

        def tdrain(i, carry2):
            pltpu.make_async_copy(polar_hbm.at[pl.ds(0, _WW)],
                                  tables_v[0].at[pl.ds(0, _WW)], sem_tab).wait()
            return carry2
        lax.fori_loop(0, _PPASS * _POLAR[0], tdrain, 0)

        def bdrain(i, carry2):
            pltpu.make_async_copy(ref_hbm.at[pl.ds(0, _BOXW)],
                                  tables_v[0].at[pl.ds(0, _BOXW)], sem_tab).wait()
            return carry2
        lax.fori_loop(0, _PPASS * _BOXW, bdrain, 0)

        def chunk_pair_body(half, carry2):
            for b in range(2):
                cix = half * 2 + b
                wait_in(b)
                pl.when(cix >= 2)(lambda b=b: drain_out(b))
                inbuf = inbufs_v[b]
                y0 = qr0 + cix * _ROWS
                for t in range(_PPASS):
                    table_v = tables_v[t]
                    ob_v = obufs_v[t][b]

                    @plsc.parallel_loop(0, _K, 16, unroll=8)
                    def px_body(o):
                        idx = inbuf[pl.ds(o, 16)]
                        w = inbuf[pl.ds(_K + o, 16)]
                        v00 = plsc.load_gather(table_v, [idx])
                        v01 = plsc.load_gather(table_v, [idx + 1])
                        v10 = plsc.load_gather(table_v, [idx + _WW])
                        v11 = plsc.load_gather(table_v, [idx + (_WW + 1)])
                        wx0 = plsc.bitcast(
                            jnp.bitwise_and(w, jnp.int32(-65536)), jnp.float32)
                        wy0 = plsc.bitcast(lax.shift_left(w, 16), jnp.float32)
                        wx1 = 1.0 - wx0
                        wy1 = 1.0 - wy0
                        r = (wy0 * (wx0 * v00 + wx1 * v01)
                             + wy1 * (wx0 * v10 + wx1 * v11))
                        ob_v[o // 256, pl.ds(o % 256, 16)] = r

                    pltpu.async_copy(
                        ob_v,
                        out_hbm.at[pbase + t, pl.ds(y0, _ROWS), pl.ds(qc0, 256)],
                        sem_out[b])
                pl.when(cix + 2 < _NCHUNK)(lambda cix=cix, b=b: start_in(cix + 2, b))
            return carry2

        lax.fori_loop(0, _NCHUNK // 2, chunk_pair_body, 0)
        drain_out(0)
        drain_out(1)
        return carry

    lax.fori_loop(0, _NPASS, pass_body, 0)


def kernel(polar_feat, ref_feat):
    polar1 = polar_feat.reshape(-1)
    ref1 = ref_feat.reshape(-1)
    stream = jnp.asarray(_STREAM_NP)
    out = _make_sc_resample()(polar1, ref1, stream)
    return out.reshape(_B, _C, _CART[0], _CART[1])
